# baseline (speedup 1.0000x reference)
import functools

import jax
import jax.numpy as jnp
from jax import lax
from jax.experimental import pallas as pl
from jax.experimental.pallas import tpu as pltpu
from jax.experimental.pallas import tpu_sc as plsc

HIDDEN = 64
CH = 128
NC = 2
NS = 16
NW = NC * NS


def _gather(table, idx3d):
    rows_per_w = idx3d.shape[1]
    n = NW * rows_per_w * CH

    mesh = plsc.VectorSubcoreMesh(
        core_axis_name="c", subcore_axis_name="s", num_cores=NC, num_subcores=NS
    )

    @functools.partial(
        pl.kernel,
        out_type=jax.ShapeDtypeStruct((n, HIDDEN), jnp.float32),
        mesh=mesh,
        scratch_types=[
            pltpu.VMEM((rows_per_w, CH), jnp.int32),
            pltpu.VMEM((CH, HIDDEN), jnp.float32),
            pltpu.SemaphoreType.DMA,
        ],
        compiler_params=pltpu.CompilerParams(use_tc_tiling_on_sc=False),
    )
    def k(table_hbm, idx_hbm, out_hbm, idx_v, rows_v, sem):
        wid = lax.axis_index("s") * NC + lax.axis_index("c")
        row0 = wid * rows_per_w
        pltpu.sync_copy(idx_hbm.at[wid], idx_v)

        def step(i, carry):
            pltpu.async_copy(table_hbm.at[idx_v.at[i]], rows_v, sem).wait()
            pltpu.sync_copy(rows_v, out_hbm.at[pl.ds((row0 + i) * CH, CH)])
            return carry

        lax.fori_loop(0, rows_per_w, step, 0)

    return k(table, idx3d)


def kernel(token_ids, key, embed_table):
    b, s = token_ids.shape
    flat = jnp.reshape(token_ids.astype(jnp.int32), (NW, b * s // (NW * CH), CH))
    out = _gather(embed_table, flat)
    return jnp.reshape(out, (b, s, HIDDEN))


# speedup vs baseline: 1.0469x; 1.0469x over previous
import functools

import jax
import jax.numpy as jnp
from jax import lax
from jax.experimental import pallas as pl
from jax.experimental.pallas import tpu as pltpu
from jax.experimental.pallas import tpu_sc as plsc

HIDDEN = 64
CH = 128
NC = 2
NS = 16
NW = NC * NS


NBUF = 10


def _gather(table, idx3d):
    rows_per_w = idx3d.shape[1]
    n = NW * rows_per_w * CH
    ngroups = rows_per_w // NBUF

    mesh = plsc.VectorSubcoreMesh(
        core_axis_name="c", subcore_axis_name="s", num_cores=NC, num_subcores=NS
    )

    @functools.partial(
        pl.kernel,
        out_type=jax.ShapeDtypeStruct((n, HIDDEN), jnp.float32),
        mesh=mesh,
        scratch_types=[
            pltpu.VMEM((rows_per_w, CH), jnp.int32),
            pltpu.VMEM((NBUF, CH, HIDDEN), jnp.float32),
            pltpu.SemaphoreType.DMA((NBUF,)),
            pltpu.SemaphoreType.DMA((NBUF,)),
        ],
        compiler_params=pltpu.CompilerParams(use_tc_tiling_on_sc=False),
    )
    def k(table_hbm, idx_hbm, out_hbm, idx_v, rows_v, gsem, wsem):
        wid = lax.axis_index("s") * NC + lax.axis_index("c")
        row0 = wid * rows_per_w
        pltpu.sync_copy(idx_hbm.at[wid], idx_v)

        def fire_gather(chunk, b):
            pltpu.async_copy(table_hbm.at[idx_v.at[chunk]], rows_v.at[b], gsem.at[b])

        def fire_write(chunk, b):
            pltpu.async_copy(
                rows_v.at[b], out_hbm.at[pl.ds((row0 + chunk) * CH, CH)], wsem.at[b]
            )

        def wait_gather(b):
            pltpu.make_async_copy(table_hbm.at[pl.ds(0, CH)], rows_v.at[b], gsem.at[b]).wait()

        def wait_write(chunk, b):
            pltpu.make_async_copy(
                rows_v.at[b], out_hbm.at[pl.ds((row0 + chunk) * CH, CH)], wsem.at[b]
            ).wait()

        for b in range(NBUF):
            fire_gather(b, b)

        def group(g, carry):
            for b in range(NBUF):
                wait_gather(b)
                fire_write(g * NBUF + b, b)
            for b in range(NBUF):
                nxt = (g + 1) * NBUF + b

                @pl.when(g + 1 < ngroups)
                def _():
                    wait_write(g * NBUF + b, b)
                    fire_gather(nxt, b)

            return carry

        lax.fori_loop(0, ngroups, group, 0)
        for b in range(NBUF):
            wait_write((ngroups - 1) * NBUF + b, b)

    return k(table, idx3d)


def kernel(token_ids, key, embed_table):
    b, s = token_ids.shape
    flat = jnp.reshape(token_ids.astype(jnp.int32), (NW, b * s // (NW * CH), CH))
    out = _gather(embed_table, flat)
    return jnp.reshape(out, (b, s, HIDDEN))
